# baseline (speedup 1.0000x reference)
import jax
import jax.numpy as jnp
from jax.experimental import pallas as pl
from jax.experimental.pallas import tpu as pltpu

_BM = 8192
_BK = 512


def _router_block(x_ref, w_ref, b_ref, o_ref):
    k = pl.program_id(1)
    part = jax.lax.dot_general(
        x_ref[...].astype(jnp.bfloat16),
        w_ref[...].astype(jnp.bfloat16),
        dimension_numbers=(((1,), (1,)), ((), ())),
        preferred_element_type=jnp.float32,
    )

    @pl.when(k == 0)
    def _init():
        o_ref[...] = part + b_ref[...]

    @pl.when(k != 0)
    def _acc():
        o_ref[...] += part


def kernel(x, W, b):
    n_tokens, d_model = x.shape
    n_experts = W.shape[0]
    b2 = b.reshape(1, n_experts)
    return pl.pallas_call(
        _router_block,
        grid=(n_tokens // _BM, d_model // _BK),
        in_specs=[
            pl.BlockSpec((_BM, _BK), lambda m, k: (m, k)),
            pl.BlockSpec((n_experts, _BK), lambda m, k: (0, k)),
            pl.BlockSpec((1, n_experts), lambda m, k: (0, 0)),
        ],
        out_specs=pl.BlockSpec((_BM, n_experts), lambda m, k: (m, 0)),
        out_shape=jax.ShapeDtypeStruct((n_tokens, n_experts), jnp.float32),
        compiler_params=pltpu.CompilerParams(
            vmem_limit_bytes=120 * 1024 * 1024,
            dimension_semantics=("arbitrary", "arbitrary"),
        ),
    )(x, W, b2)
